# baseline (speedup 1.0000x reference)
import jax
import jax.numpy as jnp
from jax import lax
from jax.experimental import pallas as pl
from jax.experimental.pallas import tpu as pltpu
from jax.experimental.pallas import tpu_sc as plsc

B = 16384
DIM = 64
L = 16
NC = 2
NS = 16
NW = NC * NS
BPW = B // NW
C = 128
NCH = BPW // C
NG = C // L
N_ROWS = 1000000
HALF = N_ROWS // 2
BC = 16384
BCH = BC // 2
BSH = 14
L2 = 1e-4


def _tr_body(a_ref, o_ref):
    x = a_ref[...]
    eye = jnp.eye(DIM, dtype=jnp.float32)
    dn = (((0,), (0,)), ((), ()))
    o_ref[:, 0:DIM] = lax.dot_general(
        x[:, 0:BC // 2], eye, dn, preferred_element_type=jnp.float32)
    o_ref[:, DIM:2 * DIM] = lax.dot_general(
        x[:, BC // 2:BC], eye, dn, preferred_element_type=jnp.float32)


def _tc_transpose(tT):
    nb = (N_ROWS + BC - 1) // BC
    return pl.pallas_call(
        _tr_body,
        grid=(nb,),
        in_specs=[pl.BlockSpec((DIM, BC), lambda g: (0, g))],
        out_specs=pl.BlockSpec((BC // 2, 2 * DIM), lambda g: (g, 0)),
        out_shape=jax.ShapeDtypeStruct((nb * (BC // 2), 2 * DIM),
                                       jnp.float32),
    )(tT)


def _stage_body(users_hbm, u2_hbm, out_hbm,
                idx_u, tid0, tid1, b0, b1, sem0, sem1):
    wid = lax.axis_index("s") * NC + lax.axis_index("c")
    base = wid * BPW
    pltpu.sync_copy(users_hbm.at[pl.ds(base, BPW)], idx_u)
    tids = (tid0, tid1)
    bufs = (b0, b1)
    sems = (sem0, sem1)

    def pid(iv):
        return (lax.shift_right_logical(iv, BSH) * BCH + (iv & (BCH - 1)))

    def fire(c, k):
        for b in range(NG):
            tids[k][pl.ds(b * L, L)] = pid(idx_u[pl.ds(c * C + b * L, L)])
        pltpu.async_copy(u2_hbm.at[tids[k]], bufs[k], sems[k])

    def drain(k):
        pltpu.make_async_copy(u2_hbm.at[tids[k]], bufs[k], sems[k]).wait()

    fire(0, 0)
    for c in range(NCH):
        k = c % 2
        if c + 1 < NCH:
            fire(c + 1, 1 - k)
        drain(k)
        pltpu.sync_copy(bufs[k], out_hbm.at[pl.ds(base + c * C, C)])


@jax.jit
def _sc_stage_users(users, u2):
    mesh = plsc.VectorSubcoreMesh(core_axis_name="c", subcore_axis_name="s",
                                  num_cores=NC, num_subcores=NS)
    f = pl.kernel(
        _stage_body,
        out_type=[jax.ShapeDtypeStruct((B, 2 * DIM), jnp.float32)],
        mesh=mesh,
        compiler_params=pltpu.CompilerParams(needs_layout_passes=False),
        scratch_types=[
            pltpu.VMEM((BPW,), jnp.int32),
            pltpu.VMEM((C,), jnp.int32),
            pltpu.VMEM((C,), jnp.int32),
            pltpu.VMEM((C, 2 * DIM), jnp.float32),
            pltpu.VMEM((C, 2 * DIM), jnp.float32),
            pltpu.SemaphoreType.DMA,
            pltpu.SemaphoreType.DMA,
        ],
    )
    return f(users, u2)[0]


def _sc_body(users_hbm, pos_hbm, neg_hbm, u2_hbm, i2_hbm,
             pos_out, neg_out, reg_out,
             idx_u, idx_p, idx_n,
             tid0, tid1,
             bu0, bp0, bn0, bu1, bp1, bn1,
             pos_buf, neg_buf, reg_buf, sem0, sem1):
    wid = lax.axis_index("s") * NC + lax.axis_index("c")
    base = wid * BPW

    pltpu.sync_copy(users_hbm.at[pl.ds(base, BPW)], idx_u)
    pltpu.sync_copy(pos_hbm.at[pl.ds(base, BPW)], idx_p)
    pltpu.sync_copy(neg_hbm.at[pl.ds(base, BPW)], idx_n)

    idxs = (idx_u, idx_p, idx_n)
    tabs = (u2_hbm, i2_hbm, i2_hbm)
    bufs = ((bu0, bp0, bn0), (bu1, bp1, bn1))
    tids = (tid0, tid1)
    sems = (sem0, sem1)
    jvec = lax.iota(jnp.int32, L)

    def pair_id(t, iv):
        return (lax.shift_right_logical(iv, BSH) * BCH + (iv & (BCH - 1)))

    def pair_col(t, iv):
        return (lax.shift_right_logical(iv, BSH - 1) & 1) * DIM

    def fire(c, k):
        pltpu.async_copy(u2_hbm.at[pl.ds(base + c * C, C)], bufs[k][0],
                         sems[k])
        for t in (1, 2):
            for b in range(NG):
                sl = pl.ds(c * C + b * L, L)
                tids[k][t, pl.ds(b * L, L)] = pair_id(t, idxs[t][sl])
        for t in (1, 2):
            pltpu.async_copy(tabs[t].at[tids[k].at[t]], bufs[k][t], sems[k])

    def drain(k):
        pltpu.make_async_copy(u2_hbm.at[pl.ds(base, C)], bufs[k][0],
                              sems[k]).wait()
        for t in (1, 2):
            pltpu.make_async_copy(tabs[t].at[tids[k].at[t]], bufs[k][t],
                                  sems[k]).wait()

    def compute(c, k, acc_sq):
        bu, bp, bn = bufs[k]

        def group(b, acc_sq):
            sl = pl.ds(c * C + b * L, L)
            row = b * L + jvec
            cu = pair_col(0, idx_u[sl])
            cp = pair_col(1, idx_p[sl])
            cn = pair_col(2, idx_n[sl])
            acc_p = jnp.zeros((L,), jnp.float32)
            acc_n = jnp.zeros((L,), jnp.float32)
            for d in range(DIM):
                gu = plsc.load_gather(bu, [row, cu + d])
                gp = plsc.load_gather(bp, [row, cp + d])
                gn = plsc.load_gather(bn, [row, cn + d])
                acc_p = acc_p + gu * gp
                acc_n = acc_n + gu * gn
                acc_sq = acc_sq + gu * gu
                acc_sq = acc_sq + gp * gp
                acc_sq = acc_sq + gn * gn
            pos_buf[sl] = acc_p
            neg_buf[sl] = acc_n
            return acc_sq

        return lax.fori_loop(0, NG, group, acc_sq)

    fire(0, 0)

    def step(cc, acc_sq):
        c0 = 2 * cc
        fire(c0 + 1, 1)
        drain(0)
        acc_sq = compute(c0, 0, acc_sq)

        @pl.when(cc < NCH // 2 - 1)
        def _():
            fire(c0 + 2, 0)

        drain(1)
        acc_sq = compute(c0 + 1, 1, acc_sq)
        return acc_sq

    acc_sq = lax.fori_loop(0, NCH // 2, step, jnp.zeros((L,), jnp.float32))
    reg_buf[...] = acc_sq

    pltpu.sync_copy(pos_buf, pos_out.at[pl.ds(base, BPW)])
    pltpu.sync_copy(neg_buf, neg_out.at[pl.ds(base, BPW)])
    pltpu.sync_copy(reg_buf, reg_out.at[wid])


@jax.jit
def _sc_gather_scores(users, pos_items, neg0, u2, i2):
    mesh = plsc.VectorSubcoreMesh(core_axis_name="c", subcore_axis_name="s",
                                  num_cores=NC, num_subcores=NS)
    f = pl.kernel(
        _sc_body,
        out_type=[
            jax.ShapeDtypeStruct((B,), jnp.float32),
            jax.ShapeDtypeStruct((B,), jnp.float32),
            jax.ShapeDtypeStruct((NW, L), jnp.float32),
        ],
        mesh=mesh,
        compiler_params=pltpu.CompilerParams(needs_layout_passes=False),
        scratch_types=[
            pltpu.VMEM((BPW,), jnp.int32),
            pltpu.VMEM((BPW,), jnp.int32),
            pltpu.VMEM((BPW,), jnp.int32),
            pltpu.VMEM((3, C), jnp.int32),
            pltpu.VMEM((3, C), jnp.int32),
            pltpu.VMEM((C, 2 * DIM), jnp.float32),
            pltpu.VMEM((C, 2 * DIM), jnp.float32),
            pltpu.VMEM((C, 2 * DIM), jnp.float32),
            pltpu.VMEM((C, 2 * DIM), jnp.float32),
            pltpu.VMEM((C, 2 * DIM), jnp.float32),
            pltpu.VMEM((C, 2 * DIM), jnp.float32),
            pltpu.VMEM((BPW,), jnp.float32),
            pltpu.VMEM((BPW,), jnp.float32),
            pltpu.VMEM((L,), jnp.float32),
            pltpu.SemaphoreType.DMA,
            pltpu.SemaphoreType.DMA,
        ],
    )
    return f(users, pos_items, neg0, u2, i2)


def _tc_loss_body(pos_ref, neg_ref, reg_ref, out_ref):
    x = neg_ref[...] - pos_ref[...]
    mf = jnp.sum(jnp.log(1.0 + jnp.exp(x))) / B
    reg = jnp.sum(reg_ref[...])
    out_ref[0, 0] = mf + L2 * reg / (2.0 * B)


def _tc_loss(pos2d, neg2d, reg2d):
    return pl.pallas_call(
        _tc_loss_body,
        out_shape=jax.ShapeDtypeStruct((1, 1), jnp.float32),
        out_specs=pl.BlockSpec(memory_space=pltpu.SMEM),
    )(pos2d, neg2d, reg2d)


def kernel(cur_epoch, users, pos_items, neg_items, user_emb, item_emb):
    users = users.astype(jnp.int32)
    pos_items = pos_items.astype(jnp.int32)
    neg0 = neg_items[:, 0].astype(jnp.int32)
    u2 = _tc_transpose(user_emb.T)
    u_stage = _sc_stage_users(users, u2)
    i2 = _tc_transpose(item_emb.T)
    pos_scores, neg_scores, reg = _sc_gather_scores(
        users, pos_items, neg0, u_stage, i2)
    loss = _tc_loss(pos_scores.reshape(128, 128),
                    neg_scores.reshape(128, 128),
                    reg.reshape(4, 128))[0, 0]
    return (loss, pos_scores, neg_scores.reshape(B, 1))


# speedup vs baseline: 1.0063x; 1.0063x over previous
import jax
import jax.numpy as jnp
from jax import lax
from jax.experimental import pallas as pl
from jax.experimental.pallas import tpu as pltpu
from jax.experimental.pallas import tpu_sc as plsc

B = 16384
DIM = 64
L = 16
NC = 2
NS = 16
NW = NC * NS
BPW = B // NW
C = 128
NCH = BPW // C
NG = C // L
N_ROWS = 1000000
HALF = N_ROWS // 2
BC = 16384
BCH = BC // 2
BSH = 14
L2 = 1e-4


def _tr_body(a_ref, o_ref):
    x = a_ref[...]
    eye = jnp.eye(DIM, dtype=jnp.float32)
    dn = (((0,), (0,)), ((), ()))
    o_ref[:, 0:DIM] = lax.dot_general(
        x[:, 0:BC // 2], eye, dn, preferred_element_type=jnp.float32)
    o_ref[:, DIM:2 * DIM] = lax.dot_general(
        x[:, BC // 2:BC], eye, dn, preferred_element_type=jnp.float32)


def _tc_transpose(tT):
    nb = (N_ROWS + BC - 1) // BC
    return pl.pallas_call(
        _tr_body,
        grid=(nb,),
        in_specs=[pl.BlockSpec((DIM, BC), lambda g: (0, g))],
        out_specs=pl.BlockSpec((BC // 2, 2 * DIM), lambda g: (g, 0)),
        out_shape=jax.ShapeDtypeStruct((nb * (BC // 2), 2 * DIM),
                                       jnp.float32),
    )(tT)


def _sc_body(users_hbm, pos_hbm, neg_hbm, u2_hbm, i2_hbm,
             pos_out, neg_out, reg_out,
             idx_u, idx_p, idx_n,
             tid0, tid1,
             bu0, bp0, bn0, bu1, bp1, bn1,
             pos_buf, neg_buf, reg_buf, sem0, sem1):
    wid = lax.axis_index("s") * NC + lax.axis_index("c")
    base = wid * BPW

    pltpu.sync_copy(users_hbm.at[pl.ds(base, BPW)], idx_u)
    pltpu.sync_copy(pos_hbm.at[pl.ds(base, BPW)], idx_p)
    pltpu.sync_copy(neg_hbm.at[pl.ds(base, BPW)], idx_n)

    idxs = (idx_u, idx_p, idx_n)
    tabs = (u2_hbm, i2_hbm, i2_hbm)
    bufs = ((bu0, bp0, bn0), (bu1, bp1, bn1))
    tids = (tid0, tid1)
    sems = (sem0, sem1)
    jvec = lax.iota(jnp.int32, L)

    def pair_id(t, iv):
        return (lax.shift_right_logical(iv, BSH) * BCH + (iv & (BCH - 1)))

    def pair_col(t, iv):
        return (lax.shift_right_logical(iv, BSH - 1) & 1) * DIM

    def fire(c, k):
        for t in range(3):
            for b in range(NG):
                sl = pl.ds(c * C + b * L, L)
                tids[k][t, pl.ds(b * L, L)] = pair_id(t, idxs[t][sl])
        for t in range(3):
            pltpu.async_copy(tabs[t].at[tids[k].at[t]], bufs[k][t], sems[k])

    def drain(k):
        for t in range(3):
            pltpu.make_async_copy(tabs[t].at[tids[k].at[t]], bufs[k][t],
                                  sems[k]).wait()

    def compute(c, k, acc_sq):
        bu, bp, bn = bufs[k]

        def group(b, acc_sq):
            sl = pl.ds(c * C + b * L, L)
            row = b * L + jvec
            cu = pair_col(0, idx_u[sl])
            cp = pair_col(1, idx_p[sl])
            cn = pair_col(2, idx_n[sl])
            acc_p = jnp.zeros((L,), jnp.float32)
            acc_n = jnp.zeros((L,), jnp.float32)
            for d in range(DIM):
                gu = plsc.load_gather(bu, [row, cu + d])
                gp = plsc.load_gather(bp, [row, cp + d])
                gn = plsc.load_gather(bn, [row, cn + d])
                acc_p = acc_p + gu * gp
                acc_n = acc_n + gu * gn
                acc_sq = acc_sq + gu * gu
                acc_sq = acc_sq + gp * gp
                acc_sq = acc_sq + gn * gn
            pos_buf[sl] = acc_p
            neg_buf[sl] = acc_n
            return acc_sq

        return lax.fori_loop(0, NG, group, acc_sq)

    fire(0, 0)

    def step(cc, acc_sq):
        c0 = 2 * cc
        fire(c0 + 1, 1)
        drain(0)
        acc_sq = compute(c0, 0, acc_sq)

        @pl.when(cc < NCH // 2 - 1)
        def _():
            fire(c0 + 2, 0)

        drain(1)
        acc_sq = compute(c0 + 1, 1, acc_sq)
        return acc_sq

    acc_sq = lax.fori_loop(0, NCH // 2, step, jnp.zeros((L,), jnp.float32))
    reg_buf[...] = acc_sq

    pltpu.sync_copy(pos_buf, pos_out.at[pl.ds(base, BPW)])
    pltpu.sync_copy(neg_buf, neg_out.at[pl.ds(base, BPW)])
    pltpu.sync_copy(reg_buf, reg_out.at[wid])


@jax.jit
def _sc_gather_scores(users, pos_items, neg0, u2, i2):
    mesh = plsc.VectorSubcoreMesh(core_axis_name="c", subcore_axis_name="s",
                                  num_cores=NC, num_subcores=NS)
    f = pl.kernel(
        _sc_body,
        out_type=[
            jax.ShapeDtypeStruct((B,), jnp.float32),
            jax.ShapeDtypeStruct((B,), jnp.float32),
            jax.ShapeDtypeStruct((NW, L), jnp.float32),
        ],
        mesh=mesh,
        compiler_params=pltpu.CompilerParams(needs_layout_passes=False),
        scratch_types=[
            pltpu.VMEM((BPW,), jnp.int32),
            pltpu.VMEM((BPW,), jnp.int32),
            pltpu.VMEM((BPW,), jnp.int32),
            pltpu.VMEM((3, C), jnp.int32),
            pltpu.VMEM((3, C), jnp.int32),
            pltpu.VMEM((C, 2 * DIM), jnp.float32),
            pltpu.VMEM((C, 2 * DIM), jnp.float32),
            pltpu.VMEM((C, 2 * DIM), jnp.float32),
            pltpu.VMEM((C, 2 * DIM), jnp.float32),
            pltpu.VMEM((C, 2 * DIM), jnp.float32),
            pltpu.VMEM((C, 2 * DIM), jnp.float32),
            pltpu.VMEM((BPW,), jnp.float32),
            pltpu.VMEM((BPW,), jnp.float32),
            pltpu.VMEM((L,), jnp.float32),
            pltpu.SemaphoreType.DMA,
            pltpu.SemaphoreType.DMA,
        ],
    )
    return f(users, pos_items, neg0, u2, i2)


def _tc_loss_body(pos_ref, neg_ref, reg_ref, out_ref):
    x = neg_ref[...] - pos_ref[...]
    mf = jnp.sum(jnp.log(1.0 + jnp.exp(x))) / B
    reg = jnp.sum(reg_ref[...])
    out_ref[0, 0] = mf + L2 * reg / (2.0 * B)


def _tc_loss(pos2d, neg2d, reg2d):
    return pl.pallas_call(
        _tc_loss_body,
        out_shape=jax.ShapeDtypeStruct((1, 1), jnp.float32),
        out_specs=pl.BlockSpec(memory_space=pltpu.SMEM),
    )(pos2d, neg2d, reg2d)


def kernel(cur_epoch, users, pos_items, neg_items, user_emb, item_emb):
    users = users.astype(jnp.int32)
    pos_items = pos_items.astype(jnp.int32)
    neg0 = neg_items[:, 0].astype(jnp.int32)
    u2 = _tc_transpose(user_emb.T)
    i2 = _tc_transpose(item_emb.T)
    pos_scores, neg_scores, reg = _sc_gather_scores(
        users, pos_items, neg0, u2, i2)
    loss = _tc_loss(pos_scores.reshape(128, 128),
                    neg_scores.reshape(128, 128),
                    reg.reshape(4, 128))[0, 0]
    return (loss, pos_scores, neg_scores.reshape(B, 1))


# speedup vs baseline: 1.0621x; 1.0554x over previous
import jax
import jax.numpy as jnp
from jax import lax
from jax.experimental import pallas as pl
from jax.experimental.pallas import tpu as pltpu
from jax.experimental.pallas import tpu_sc as plsc

B = 16384
DIM = 64
L = 16
NC = 2
NS = 16
NW = NC * NS
BPW = B // NW
C = 128
NCH = BPW // C
NG = C // L
N_ROWS = 1000000
HALF = N_ROWS // 2
BC = 32768
BCH = BC // 2
BSH = 15
L2 = 1e-4


def _tr_body(a_ref, o_ref):
    x = a_ref[...]
    eye = jnp.eye(DIM, dtype=jnp.float32)
    dn = (((0,), (0,)), ((), ()))
    o_ref[:, 0:DIM] = lax.dot_general(
        x[:, 0:BC // 2], eye, dn, preferred_element_type=jnp.float32)
    o_ref[:, DIM:2 * DIM] = lax.dot_general(
        x[:, BC // 2:BC], eye, dn, preferred_element_type=jnp.float32)


def _tc_transpose(tT):
    nb = (N_ROWS + BC - 1) // BC
    return pl.pallas_call(
        _tr_body,
        grid=(nb,),
        in_specs=[pl.BlockSpec((DIM, BC), lambda g: (0, g))],
        out_specs=pl.BlockSpec((BC // 2, 2 * DIM), lambda g: (g, 0)),
        out_shape=jax.ShapeDtypeStruct((nb * (BC // 2), 2 * DIM),
                                       jnp.float32),
    )(tT)


def _sc_body(users_hbm, pos_hbm, neg_hbm, u2_hbm, i2_hbm,
             pos_out, neg_out, reg_out,
             idx_u, idx_p, idx_n,
             tid0, tid1,
             bu0, bp0, bn0, bu1, bp1, bn1,
             pos_buf, neg_buf, reg_buf, sem0, sem1):
    wid = lax.axis_index("s") * NC + lax.axis_index("c")
    base = wid * BPW

    pltpu.sync_copy(users_hbm.at[pl.ds(base, BPW)], idx_u)
    pltpu.sync_copy(pos_hbm.at[pl.ds(base, BPW)], idx_p)
    pltpu.sync_copy(neg_hbm.at[pl.ds(base, BPW)], idx_n)

    idxs = (idx_u, idx_p, idx_n)
    tabs = (u2_hbm, i2_hbm, i2_hbm)
    bufs = ((bu0, bp0, bn0), (bu1, bp1, bn1))
    tids = (tid0, tid1)
    sems = (sem0, sem1)
    jvec = lax.iota(jnp.int32, L)

    def pair_id(t, iv):
        return (lax.shift_right_logical(iv, BSH) * BCH + (iv & (BCH - 1)))

    def pair_col(t, iv):
        return (lax.shift_right_logical(iv, BSH - 1) & 1) * DIM

    def fire(c, k):
        for t in range(3):
            for b in range(NG):
                sl = pl.ds(c * C + b * L, L)
                tids[k][t, pl.ds(b * L, L)] = pair_id(t, idxs[t][sl])
        for t in range(3):
            pltpu.async_copy(tabs[t].at[tids[k].at[t]], bufs[k][t], sems[k])

    def drain(k):
        for t in range(3):
            pltpu.make_async_copy(tabs[t].at[tids[k].at[t]], bufs[k][t],
                                  sems[k]).wait()

    def compute(c, k, acc_sq):
        bu, bp, bn = bufs[k]

        def group(b, acc_sq):
            sl = pl.ds(c * C + b * L, L)
            row = b * L + jvec
            cu = pair_col(0, idx_u[sl])
            cp = pair_col(1, idx_p[sl])
            cn = pair_col(2, idx_n[sl])
            acc_p = jnp.zeros((L,), jnp.float32)
            acc_n = jnp.zeros((L,), jnp.float32)
            for d in range(DIM):
                gu = plsc.load_gather(bu, [row, cu + d])
                gp = plsc.load_gather(bp, [row, cp + d])
                gn = plsc.load_gather(bn, [row, cn + d])
                acc_p = acc_p + gu * gp
                acc_n = acc_n + gu * gn
                acc_sq = acc_sq + gu * gu
                acc_sq = acc_sq + gp * gp
                acc_sq = acc_sq + gn * gn
            pos_buf[sl] = acc_p
            neg_buf[sl] = acc_n
            return acc_sq

        return lax.fori_loop(0, NG, group, acc_sq)

    fire(0, 0)

    def step(cc, acc_sq):
        c0 = 2 * cc
        fire(c0 + 1, 1)
        drain(0)
        acc_sq = compute(c0, 0, acc_sq)

        @pl.when(cc < NCH // 2 - 1)
        def _():
            fire(c0 + 2, 0)

        drain(1)
        acc_sq = compute(c0 + 1, 1, acc_sq)
        return acc_sq

    acc_sq = lax.fori_loop(0, NCH // 2, step, jnp.zeros((L,), jnp.float32))
    reg_buf[...] = acc_sq

    pltpu.sync_copy(pos_buf, pos_out.at[pl.ds(base, BPW)])
    pltpu.sync_copy(neg_buf, neg_out.at[pl.ds(base, BPW)])
    pltpu.sync_copy(reg_buf, reg_out.at[wid])


@jax.jit
def _sc_gather_scores(users, pos_items, neg0, u2, i2):
    mesh = plsc.VectorSubcoreMesh(core_axis_name="c", subcore_axis_name="s",
                                  num_cores=NC, num_subcores=NS)
    f = pl.kernel(
        _sc_body,
        out_type=[
            jax.ShapeDtypeStruct((B,), jnp.float32),
            jax.ShapeDtypeStruct((B,), jnp.float32),
            jax.ShapeDtypeStruct((NW, L), jnp.float32),
        ],
        mesh=mesh,
        compiler_params=pltpu.CompilerParams(needs_layout_passes=False),
        scratch_types=[
            pltpu.VMEM((BPW,), jnp.int32),
            pltpu.VMEM((BPW,), jnp.int32),
            pltpu.VMEM((BPW,), jnp.int32),
            pltpu.VMEM((3, C), jnp.int32),
            pltpu.VMEM((3, C), jnp.int32),
            pltpu.VMEM((C, 2 * DIM), jnp.float32),
            pltpu.VMEM((C, 2 * DIM), jnp.float32),
            pltpu.VMEM((C, 2 * DIM), jnp.float32),
            pltpu.VMEM((C, 2 * DIM), jnp.float32),
            pltpu.VMEM((C, 2 * DIM), jnp.float32),
            pltpu.VMEM((C, 2 * DIM), jnp.float32),
            pltpu.VMEM((BPW,), jnp.float32),
            pltpu.VMEM((BPW,), jnp.float32),
            pltpu.VMEM((L,), jnp.float32),
            pltpu.SemaphoreType.DMA,
            pltpu.SemaphoreType.DMA,
        ],
    )
    return f(users, pos_items, neg0, u2, i2)


def _tc_loss_body(pos_ref, neg_ref, reg_ref, out_ref):
    x = neg_ref[...] - pos_ref[...]
    mf = jnp.sum(jnp.log(1.0 + jnp.exp(x))) / B
    reg = jnp.sum(reg_ref[...])
    out_ref[0, 0] = mf + L2 * reg / (2.0 * B)


def _tc_loss(pos2d, neg2d, reg2d):
    return pl.pallas_call(
        _tc_loss_body,
        out_shape=jax.ShapeDtypeStruct((1, 1), jnp.float32),
        out_specs=pl.BlockSpec(memory_space=pltpu.SMEM),
    )(pos2d, neg2d, reg2d)


def kernel(cur_epoch, users, pos_items, neg_items, user_emb, item_emb):
    users = users.astype(jnp.int32)
    pos_items = pos_items.astype(jnp.int32)
    neg0 = neg_items[:, 0].astype(jnp.int32)
    u2 = _tc_transpose(user_emb.T)
    i2 = _tc_transpose(item_emb.T)
    pos_scores, neg_scores, reg = _sc_gather_scores(
        users, pos_items, neg0, u2, i2)
    loss = _tc_loss(pos_scores.reshape(128, 128),
                    neg_scores.reshape(128, 128),
                    reg.reshape(4, 128))[0, 0]
    return (loss, pos_scores, neg_scores.reshape(B, 1))
